# Initial kernel scaffold; baseline (speedup 1.0000x reference)
import jax
import jax.numpy as jnp
from jax.experimental import pallas as pl


def kernel(h, code_emb, u, conf_mask, fc_W1, fc_b1, fc_W2, fc_b2, sc_W1, sc_b1, sc_W2, sc_b2, sh_W1, sh_b1, sh_W2, sh_b2, pj_W, pj_b, rt_W, rt_b, ex_W, ex_b):
    raise NotImplementedError("write your pallas kernel here")


# speedup vs baseline: 1.4311x; 1.4311x over previous
import jax
import jax.numpy as jnp
from jax.experimental import pallas as pl
from jax.experimental.pallas import tpu as pltpu

N = 2048
HID = 1024
CODE = 256
CONF = 64
E = 8
BLK = 256
NBLK = N // BLK

_f32 = jnp.float32
_bf16 = jnp.bfloat16


def _dot(a, b):
    return jax.lax.dot_general(a, b, (((1,), (0,)), ((), ())),
                               preferred_element_type=_f32)


def _mega_body(code_ref, u_ref, h_ref,
               scW1c_ref, scW1u_ref, scW2_ref,
               shW1c_ref, shW1u_ref, shW2_ref,
               pj_ref, fcW1_ref, fcW2_ref, exT_ref, rtT_ref,
               sc_b1_ref, sc_b2_ref, sh_b1_ref, sh_b2_ref,
               pj_b_ref, fc_b1_ref, fc_b2_ref, rt_b_ref, ex_b_ref,
               cmask_ref,
               out_ref, aux_ref, acc_ref):
    i = pl.program_id(0)
    code = code_ref[...]
    u = u_ref[...]

    t = _dot(code, scW1c_ref[...]) + _dot(u, scW1u_ref[...]) + sc_b1_ref[...]
    t = t * jax.nn.sigmoid(t)
    scale = jax.nn.sigmoid(_dot(t.astype(_bf16), scW2_ref[...]) + sc_b2_ref[...])

    s = _dot(code, shW1c_ref[...]) + _dot(u, shW1u_ref[...]) + sh_b1_ref[...]
    s = s * jax.nn.sigmoid(s)
    shift = _dot(s.astype(_bf16), shW2_ref[...]) + sh_b2_ref[...]

    ht = jnp.maximum(_dot(h_ref[...], pj_ref[...]) + pj_b_ref[...], 0.0)
    hm = (scale * ht + shift).astype(_bf16)

    logits = _dot(code, rtT_ref[...]) + rt_b_ref[...]
    mx = jnp.max(logits, axis=1, keepdims=True)
    ex_ = jnp.exp(logits - mx)
    probs = ex_ / jnp.sum(ex_, axis=1, keepdims=True)
    lane = jax.lax.broadcasted_iota(jnp.int32, (BLK, E), 1)
    m1 = jnp.max(probs, axis=1, keepdims=True)
    e1 = jnp.min(jnp.where(probs >= m1, lane, E), axis=1, keepdims=True)
    p_no1 = jnp.where(lane == e1, -1.0, probs)
    m2 = jnp.max(p_no1, axis=1, keepdims=True)
    maskD = (probs >= m2).astype(_f32)
    g = probs * maskD
    g = g / (jnp.sum(g, axis=1, keepdims=True) + 1e-9)

    @pl.when(i == 0)
    def _():
        acc_ref[...] = jnp.zeros_like(acc_ref)

    acc_ref[0:1, :] += jnp.sum(probs, axis=0, keepdims=True)
    acc_ref[1:2, :] += jnp.sum(maskD, axis=0, keepdims=True)

    acc = jnp.zeros((BLK, HID), _f32)
    for e in range(E):
        r = jnp.maximum(_dot(hm, exT_ref[e]) + ex_b_ref[e:e + 1, :], 0.0)
        acc += g[:, e:e + 1] * r

    cm = cmask_ref[...]
    ce = _dot(jnp.maximum(_dot(u, fcW1_ref[...]) + fc_b1_ref[...], 0.0).astype(_bf16),
              fcW2_ref[...]) + fc_b2_ref[...]
    ce = ce * (cm > 0.0).astype(_f32)
    out_ref[...] = acc * (1.0 - jax.nn.sigmoid(cm)) + ce

    @pl.when(i == NBLK - 1)
    def _():
        pm = acc_ref[0:1, :] * (1.0 / N)
        mm = acc_ref[1:2, :] * (1.0 / N)
        aux_ref[...] = E * jnp.sum(pm * mm, axis=(0, 1), keepdims=True)


def kernel(h, code_emb, u, conf_mask, fc_W1, fc_b1, fc_W2, fc_b2,
           sc_W1, sc_b1, sc_W2, sc_b2, sh_W1, sh_b1, sh_W2, sh_b2,
           pj_W, pj_b, rt_W, rt_b, ex_W, ex_b):
    bf = _bf16
    row = lambda v: v.reshape(1, -1).astype(_f32)
    args = (
        code_emb.astype(bf), u.astype(bf), h.astype(bf),
        sc_W1[:, :CODE].T.astype(bf), sc_W1[:, CODE:].T.astype(bf),
        sc_W2.T.astype(bf),
        sh_W1[:, :CODE].T.astype(bf), sh_W1[:, CODE:].T.astype(bf),
        sh_W2.T.astype(bf),
        pj_W.T.astype(bf), fc_W1.T.astype(bf), fc_W2.T.astype(bf),
        ex_W.transpose(0, 2, 1).astype(bf), rt_W.T.astype(bf),
        row(sc_b1), row(sc_b2), row(sh_b1), row(sh_b2),
        row(pj_b), row(fc_b1), row(fc_b2), row(rt_b), ex_b.astype(_f32),
        row(conf_mask),
    )

    blk = lambda shape, im: pl.BlockSpec(shape, im)
    tok = lambda d: blk((BLK, d), lambda i: (i, 0))
    cst = lambda shape: blk(shape, lambda i: tuple(0 for _ in shape))

    in_specs = [
        tok(CODE), tok(CONF), tok(HID),
        cst((CODE, HID)), cst((CONF, HID)), cst((HID, HID)),
        cst((CODE, HID)), cst((CONF, HID)), cst((HID, HID)),
        cst((HID, HID)), cst((CONF, HID)), cst((HID, HID)),
        cst((E, HID, HID)), cst((CODE, E)),
        cst((1, HID)), cst((1, HID)), cst((1, HID)), cst((1, HID)),
        cst((1, HID)), cst((1, HID)), cst((1, HID)), cst((1, E)),
        cst((E, HID)),
        cst((1, HID)),
    ]

    out, aux = pl.pallas_call(
        _mega_body,
        grid=(NBLK,),
        in_specs=in_specs,
        out_specs=[tok(HID), cst((1, 1))],
        out_shape=[jax.ShapeDtypeStruct((N, HID), _f32),
                   jax.ShapeDtypeStruct((1, 1), _f32)],
        scratch_shapes=[pltpu.VMEM((2, E), _f32)],
    )(*args)
    return out, aux.reshape(())
